# Initial kernel scaffold; baseline (speedup 1.0000x reference)
import functools

import jax
import jax.numpy as jnp
from jax import lax
from jax.experimental import pallas as pl
from jax.experimental.pallas import tpu as pltpu
from jax.experimental.pallas import tpu_sc as plsc

N_NODES = 10000
D = 256
E = 160000
NC = 2
NS = 16
NW = NC * NS
B_PER_W = E // NW
W = 200
NCHUNK = B_PER_W // W
LANES = 16


def _dot_kernel(table_hbm, src_hbm, dst_hbm, out_hbm,
                sidx_v, didx_v, arows, brows, outv, sem_a, sem_b):
    wid = lax.axis_index("s") * NC + lax.axis_index("c")
    base = wid * B_PER_W
    pltpu.sync_copy(src_hbm.at[pl.ds(base, B_PER_W)], sidx_v)
    pltpu.sync_copy(dst_hbm.at[pl.ds(base, B_PER_W)], didx_v)

    @pl.loop(0, NCHUNK)
    def _chunk(k):
        off = k * W
        cp_a = pltpu.async_copy(
            table_hbm.at[sidx_v.at[pl.ds(off, W)]], arows, sem_a)
        cp_b = pltpu.async_copy(
            table_hbm.at[didx_v.at[pl.ds(off, W)]], brows, sem_b)
        cp_a.wait()
        cp_b.wait()

        @pl.loop(0, W)
        def _edge(w):
            acc = arows[w, pl.ds(0, LANES)] * brows[w, pl.ds(0, LANES)]
            for c in range(1, D // LANES):
                acc = acc + (arows[w, pl.ds(c * LANES, LANES)]
                             * brows[w, pl.ds(c * LANES, LANES)])
            outv[w] = jnp.sum(acc)

        pltpu.sync_copy(outv, out_hbm.at[pl.ds(base + off, W)])


@jax.jit
def kernel(node_representations, edge_index):
    src = edge_index[0].astype(jnp.int32)
    dst = edge_index[1].astype(jnp.int32)

    mesh = plsc.VectorSubcoreMesh(core_axis_name="c", subcore_axis_name="s")
    k = functools.partial(
        pl.kernel,
        mesh=mesh,
        out_type=jax.ShapeDtypeStruct((E,), jnp.float32),
        scratch_types=[
            pltpu.VMEM((B_PER_W,), jnp.int32),
            pltpu.VMEM((B_PER_W,), jnp.int32),
            pltpu.VMEM((W, D), jnp.float32),
            pltpu.VMEM((W, D), jnp.float32),
            pltpu.VMEM((W,), jnp.float32),
            pltpu.SemaphoreType.DMA,
            pltpu.SemaphoreType.DMA,
        ],
    )(_dot_kernel)
    logits = k(node_representations, src, dst)
    return logits.reshape(E, 1)


# speedup vs baseline: 2.0415x; 2.0415x over previous
import dataclasses
import functools

import jax
import jax.numpy as jnp
from jax import lax
from jax.experimental import pallas as pl
from jax.experimental.pallas import tpu as pltpu
from jax.experimental.pallas import tpu_sc as plsc

N_NODES = 10000
D = 256
E = 160000
NC = 2
NS = 16
NW = NC * NS
B_PER_W = E // NW
W = 200
NCHUNK = B_PER_W // W
LANES = 16


def _dot_kernel(table_hbm, src_hbm, dst_hbm, out_hbm,
                sidx_v, didx_v, arows, brows, outv, sem_a, sem_b):
    wid = lax.axis_index("s") * NC + lax.axis_index("c")
    base = wid * B_PER_W
    pltpu.sync_copy(src_hbm.at[pl.ds(base, B_PER_W)], sidx_v)
    pltpu.sync_copy(dst_hbm.at[pl.ds(base, B_PER_W)], didx_v)

    lane = lax.iota(jnp.int32, LANES)

    def _edge_dot(w):
        acc = arows[w, pl.ds(0, LANES)] * brows[w, pl.ds(0, LANES)]
        for c in range(1, D // LANES):
            acc = acc + (arows[w, pl.ds(c * LANES, LANES)]
                         * brows[w, pl.ds(c * LANES, LANES)])
        return jnp.sum(acc)

    @pl.loop(0, NCHUNK)
    def _chunk(k):
        off = k * W
        cp_a = pltpu.async_copy(
            table_hbm.at[sidx_v.at[pl.ds(off, W)]], arows, sem_a)
        cp_b = pltpu.async_copy(
            table_hbm.at[didx_v.at[pl.ds(off, W)]], brows, sem_b)
        cp_a.wait()
        cp_b.wait()

        @pl.loop(0, W // LANES)
        def _group(g):
            res = jnp.zeros((LANES,), jnp.float32)
            for j in range(LANES):
                res = jnp.where(lane == j, _edge_dot(g * LANES + j), res)
            outv[pl.ds(g * LANES, LANES)] = res

        n_tail = W % LANES
        if n_tail:
            res = jnp.zeros((LANES,), jnp.float32)
            for j in range(n_tail):
                res = jnp.where(lane == j, _edge_dot((W // LANES) * LANES + j),
                                res)
            outv[pl.ds((W // LANES) * LANES, LANES)] = res

        pltpu.sync_copy(outv.at[pl.ds(0, W)], out_hbm.at[pl.ds(base + off, W)])


@jax.jit
def kernel(node_representations, edge_index):
    src = edge_index[0].astype(jnp.int32)
    dst = edge_index[1].astype(jnp.int32)

    mesh = plsc.VectorSubcoreMesh(core_axis_name="c", subcore_axis_name="s")
    cp = pltpu.CompilerParams()
    if "needs_layout_passes" in pltpu.CompilerParams.__dataclass_fields__:
        cp = dataclasses.replace(cp, needs_layout_passes=False)
    k = functools.partial(
        pl.kernel,
        mesh=mesh,
        compiler_params=cp,
        out_type=jax.ShapeDtypeStruct((E,), jnp.float32),
        scratch_types=[
            pltpu.VMEM((B_PER_W,), jnp.int32),
            pltpu.VMEM((B_PER_W,), jnp.int32),
            pltpu.VMEM((W, D), jnp.float32),
            pltpu.VMEM((W, D), jnp.float32),
            pltpu.VMEM((W + (-W) % LANES, ), jnp.float32),
            pltpu.SemaphoreType.DMA,
            pltpu.SemaphoreType.DMA,
        ],
    )(_dot_kernel)
    logits = k(node_representations, src, dst)
    return logits.reshape(E, 1)
